# Initial kernel scaffold; baseline (speedup 1.0000x reference)
import functools

import numpy as np
import jax
import jax.numpy as jnp
from jax import lax
from jax.experimental import pallas as pl
from jax.experimental.pallas import tpu as pltpu
from jax.experimental.pallas import tpu_sc as plsc

_N = 10000
_E = 160000
_R = 16
_MU = np.linspace(0.0, 20.0, _R).astype(np.float32)
_INV_SIG = np.float32(_R / 20.0)

_NODE_PAIRS = [(1, 0), (1, 2), (1, 3), (0, 2), (0, 3), (3, 2)]
_EDGE_PAIRS = [(1, 1), (1, 2), (2, 1), (1, 0), (0, 1), (1, 3), (3, 1),
               (2, 2), (2, 0), (0, 2), (2, 3), (3, 2), (0, 0), (0, 3),
               (3, 0), (3, 3)]
_DXN_ATOMS = [1, 0, 2, 3]


def _mk(shape, entries):
    m = np.zeros(shape, np.float32)
    for r, c in entries:
        m[r, c] = 1.0
    return m


_R3 = range(3)
_PA = _mk((32, 48), [(3 * p + c, 3 * t + c)
                     for t, (p, q) in enumerate(_EDGE_PAIRS) for c in _R3])
_PB = _mk((16, 48), [(3 * q + c, 3 * t + c)
                     for t, (p, q) in enumerate(_EDGE_PAIRS) for c in _R3])
_PSUM = _mk((48, 16), [(3 * t + c, t) for t in range(16) for c in _R3])
_PD = _mk((16, 12), [(3 * a + c, 3 * i + c)
                     for i, a in enumerate(_DXN_ATOMS) for c in _R3])
_PN0 = _mk((32, 12), [(c, 3 * i + c) for i in range(4) for c in _R3])
_PQE = _mk((32, 36), [(12 + 3 * r + c, 9 * i + 3 * r + c)
                      for i in range(4) for r in _R3 for c in _R3])
_PDE = _mk((12, 36), [(3 * i + c, 9 * i + 3 * r + c)
                      for i in range(4) for r in _R3 for c in _R3])
_PSU = _mk((36, 16), [(9 * i + 3 * r + c, 3 * i + r)
                      for i in range(4) for r in _R3 for c in _R3])
_PNRM = _mk((12, 4), [(3 * i + c, i) for i in range(4) for c in _R3])
_PREP3 = _mk((4, 16), [(i, 3 * i + c) for i in range(4) for c in _R3])
_PREP16 = _mk((16, 256), [(t, 16 * t + k) for t in range(16) for k in range(16)])
_MUT16 = np.tile(_MU, 16).reshape(1, 256)

_NA = _mk((12, 18), [(3 * p + c, 3 * t + c)
                     for t, (p, q) in enumerate(_NODE_PAIRS) for c in _R3])
_NB = _mk((12, 18), [(3 * q + c, 3 * t + c)
                     for t, (p, q) in enumerate(_NODE_PAIRS) for c in _R3])
_NSUM = _mk((18, 6), [(3 * t + c, t) for t in range(6) for c in _R3])
_NREP = _mk((6, 96), [(t, 16 * t + k) for t in range(6) for k in range(16)])
_MUT6 = np.tile(_MU, 6).reshape(1, 96)
_DU0 = _mk((12, 3), [(3 + c, c) for c in _R3]) - _mk((12, 3), [(c, c) for c in _R3])
_DU1 = _mk((12, 3), [(6 + c, c) for c in _R3]) - _mk((12, 3), [(3 + c, c) for c in _R3])
_S1 = _mk((3, 3), [(1, 0), (2, 1), (0, 2)])
_S2 = _mk((3, 3), [(2, 0), (0, 1), (1, 2)])
_PB1 = _mk((3, 9), [(r, 3 * r) for r in _R3])
_PN9 = _mk((3, 9), [(r, 3 * r + 1) for r in _R3])
_PC9 = _mk((3, 9), [(r, 3 * r + 2) for r in _R3])
_E12T = _mk((12, 32), [(i, i) for i in range(12)])
_E9T = _mk((9, 32), [(j, 12 + j) for j in range(9)])
_E12X = _mk((12, 16), [(i, i) for i in range(12)])

_NODE_BLK = 1000
_EDGE_BLK = 1000

_CH = 128
_NROWS = _E // _CH
_NW = 32
_NJ = (_NROWS + _NW - 1) // _NW


def _hidot(a, b):
    return lax.dot_general(a, b, (((1,), (0,)), ((), ())),
                           precision=lax.Precision.HIGHEST,
                           preferred_element_type=jnp.float32)


def _cross(a, b):
    return _hidot(a, _S1) * _hidot(b, _S2) - _hidot(a, _S2) * _hidot(b, _S1)


def _nrm3(v):
    n = jnp.sqrt(jnp.sum(v * v, axis=-1, keepdims=True))
    return v / jnp.where(n == 0.0, 1.0, n)


def _node_body(x_ref, wn_ref, bn_ref, g_ref, bb_ref, hv_ref, ts_ref, tx_ref):
    X = x_ref[...]
    df = _hidot(X, _NA) - _hidot(X, _NB)
    d = jnp.sqrt(_hidot(df * df, _NSUM))
    dr = _hidot(d, _NREP)
    rb = jnp.exp(-(((dr - _MUT6) * _INV_SIG) ** 2))
    z = jnp.dot(rb, wn_ref[...], preferred_element_type=jnp.float32) + bn_ref[...]
    mu = jnp.mean(z, axis=-1, keepdims=True)
    xc = z - mu
    sig = jnp.sqrt(jnp.sum(xc * xc, axis=-1, keepdims=True) * (1.0 / 127.0) + 1e-6)
    hv_ref[...] = g_ref[...] * xc / (sig + 1e-6) + bb_ref[...]

    u0 = _nrm3(_hidot(X, _DU0))
    u1 = _nrm3(_hidot(X, _DU1))
    n0 = _nrm3(_cross(u0, u1))
    b1 = _nrm3(u0 - u1)
    cx = _cross(b1, n0)
    q9 = _hidot(b1, _PB1) + _hidot(n0, _PN9) + _hidot(cx, _PC9)
    gid = _NODE_BLK * pl.program_id(0) + lax.broadcasted_iota(
        jnp.int32, (_NODE_BLK, 1), 0)
    q9 = jnp.where(gid < _N - 1, q9, 0.0)
    ts_ref[...] = _hidot(X, _E12T) + _hidot(q9, _E9T)
    tx_ref[...] = _hidot(X, _E12X)


def _node_call(x12, wn, bn, gn, bbn):
    f32 = jnp.float32
    return pl.pallas_call(
        _node_body,
        grid=(_N // _NODE_BLK,),
        in_specs=[
            pl.BlockSpec((_NODE_BLK, 12), lambda i: (i, 0)),
            pl.BlockSpec((96, 128), lambda i: (0, 0)),
            pl.BlockSpec((1, 128), lambda i: (0, 0)),
            pl.BlockSpec((1, 128), lambda i: (0, 0)),
            pl.BlockSpec((1, 128), lambda i: (0, 0)),
        ],
        out_specs=[
            pl.BlockSpec((_NODE_BLK, 128), lambda i: (i, 0)),
            pl.BlockSpec((_NODE_BLK, 32), lambda i: (i, 0)),
            pl.BlockSpec((_NODE_BLK, 16), lambda i: (i, 0)),
        ],
        out_shape=[
            jax.ShapeDtypeStruct((_N, 128), f32),
            jax.ShapeDtypeStruct((_N, 32), f32),
            jax.ShapeDtypeStruct((_N, 16), f32),
        ],
    )(x12, wn, bn, gn, bbn)


def _edge_body(s_ref, d_ref, wd_ref, wo_ref, be_ref, g_ref, bb_ref, o_ref):
    S = s_ref[...]
    Dt = d_ref[...]
    df = _hidot(S, _PA) - _hidot(Dt, _PB)
    dist = jnp.sqrt(_hidot(df * df, _PSUM) + 1e-6)
    dr = _hidot(dist, _PREP16)
    rb = jnp.exp(-(((dr - _MUT16) * _INV_SIG) ** 2))
    dxn = _hidot(Dt, _PD) - _hidot(S, _PN0)
    du = _hidot(_hidot(S, _PQE) * _hidot(dxn, _PDE), _PSU)
    n = jnp.sqrt(_hidot(du * du, _PNRM))
    rinv = 1.0 / jnp.where(n == 0.0, 1.0, n)
    edir = du * _hidot(rinv, _PREP3)
    z = (jnp.dot(rb, wd_ref[...], preferred_element_type=jnp.float32)
         + jnp.dot(edir, wo_ref[...], preferred_element_type=jnp.float32)
         + be_ref[...])
    mu = jnp.mean(z, axis=-1, keepdims=True)
    xc = z - mu
    sig = jnp.sqrt(jnp.sum(xc * xc, axis=-1, keepdims=True) * (1.0 / 255.0) + 1e-6)
    o_ref[...] = g_ref[...] * xc / (sig + 1e-6) + bb_ref[...]


def _edge_call(s_rows, d_rows, wd, wo, be, ge, bbe):
    return pl.pallas_call(
        _edge_body,
        grid=(_E // _EDGE_BLK,),
        in_specs=[
            pl.BlockSpec((_EDGE_BLK, 32), lambda i: (i, 0)),
            pl.BlockSpec((_EDGE_BLK, 16), lambda i: (i, 0)),
            pl.BlockSpec((256, 256), lambda i: (0, 0)),
            pl.BlockSpec((16, 256), lambda i: (0, 0)),
            pl.BlockSpec((1, 256), lambda i: (0, 0)),
            pl.BlockSpec((1, 256), lambda i: (0, 0)),
            pl.BlockSpec((1, 256), lambda i: (0, 0)),
        ],
        out_specs=pl.BlockSpec((_EDGE_BLK, 256), lambda i: (i, 0)),
        out_shape=jax.ShapeDtypeStruct((_E, 256), jnp.float32),
    )(s_rows, d_rows, wd, wo, be, ge, bbe)


def _sc_gather(ts, tx, ii_s, ii_d, ie_s, ie_d):
    f32 = jnp.float32
    mesh = plsc.VectorSubcoreMesh(core_axis_name="c", subcore_axis_name="s")
    out_type = [
        jax.ShapeDtypeStruct((_E, 32), f32),
        jax.ShapeDtypeStruct((_E, 16), f32),
        jax.ShapeDtypeStruct((_E, 32), f32),
        jax.ShapeDtypeStruct((_E, 16), f32),
    ]

    @functools.partial(
        pl.kernel,
        out_type=out_type,
        mesh=mesh,
        scratch_types=[
            pltpu.VMEM((_CH,), jnp.int32),
            pltpu.VMEM((_CH, 32), f32),
            pltpu.VMEM((_CH, 16), f32),
            pltpu.SemaphoreType.DMA,
        ],
    )
    def k(ts_h, tx_h, iis_h, iid_h, ies_h, ied_h,
          o_is, o_id, o_es, o_ed, idx_v, row_s, row_d, sem):
        wid = lax.axis_index("s") * 2 + lax.axis_index("c")

        def body(j, carry):
            row = wid * _NJ + j

            @pl.when(row < _NROWS)
            def _():
                base = row * _CH
                pltpu.sync_copy(iis_h.at[row], idx_v)
                pltpu.async_copy(ts_h.at[idx_v], row_s, sem).wait()
                pltpu.sync_copy(row_s, o_is.at[pl.ds(base, _CH)])
                pltpu.sync_copy(iid_h.at[row], idx_v)
                pltpu.async_copy(tx_h.at[idx_v], row_d, sem).wait()
                pltpu.sync_copy(row_d, o_id.at[pl.ds(base, _CH)])
                pltpu.sync_copy(ies_h.at[row], idx_v)
                pltpu.async_copy(ts_h.at[idx_v], row_s, sem).wait()
                pltpu.sync_copy(row_s, o_es.at[pl.ds(base, _CH)])
                pltpu.sync_copy(ied_h.at[row], idx_v)
                pltpu.async_copy(tx_h.at[idx_v], row_d, sem).wait()
                pltpu.sync_copy(row_d, o_ed.at[pl.ds(base, _CH)])

            return carry

        lax.fori_loop(0, _NJ, body, 0)

    return k(ts, tx, ii_s, ii_d, ie_s, ie_d)


def kernel(X, E_in_idx, E_ex_idx, W_node, b_node, W_edge, b_edge,
           gain_nodes, bias_nodes, gain_edges, bias_edges):
    f32 = jnp.float32
    x12 = X.reshape(_N, 12)
    h_V, ts, tx = _node_call(x12, W_node, b_node.reshape(1, -1),
                             gain_nodes.reshape(1, -1), bias_nodes.reshape(1, -1))
    g_is, g_id, g_es, g_ed = _sc_gather(
        ts, tx,
        E_in_idx[0].reshape(_NROWS, _CH), E_in_idx[1].reshape(_NROWS, _CH),
        E_ex_idx[0].reshape(_NROWS, _CH), E_ex_idx[1].reshape(_NROWS, _CH))
    wd = W_edge[:256]
    wo = jnp.concatenate([W_edge[256:], jnp.zeros((4, 256), f32)], axis=0)
    be = b_edge.reshape(1, -1)
    ge = gain_edges.reshape(1, -1)
    bbe = bias_edges.reshape(1, -1)
    h_E_in = _edge_call(g_is, g_id, wd, wo, be, ge, bbe)
    h_E_ex = _edge_call(g_es, g_ed, wd, wo, be, ge, bbe)
    return h_V, h_E_in, h_E_ex


# speedup vs baseline: 2.5393x; 2.5393x over previous
import functools

import numpy as np
import jax
import jax.numpy as jnp
from jax import lax
from jax.experimental import pallas as pl
from jax.experimental.pallas import tpu as pltpu
from jax.experimental.pallas import tpu_sc as plsc

_N = 10000
_E = 160000
_R = 16
_MU = np.linspace(0.0, 20.0, _R).astype(np.float32)
_INV_SIG = np.float32(_R / 20.0)

_NODE_PAIRS = [(1, 0), (1, 2), (1, 3), (0, 2), (0, 3), (3, 2)]
_EDGE_PAIRS = [(1, 1), (1, 2), (2, 1), (1, 0), (0, 1), (1, 3), (3, 1),
               (2, 2), (2, 0), (0, 2), (2, 3), (3, 2), (0, 0), (0, 3),
               (3, 0), (3, 3)]
_DXN_ATOMS = [1, 0, 2, 3]


def _mk(shape, entries):
    m = np.zeros(shape, np.float32)
    for r, c in entries:
        m[r, c] = 1.0
    return m


_R3 = range(3)
_PA = _mk((32, 48), [(3 * p + c, 3 * t + c)
                     for t, (p, q) in enumerate(_EDGE_PAIRS) for c in _R3])
_PB = _mk((16, 48), [(3 * q + c, 3 * t + c)
                     for t, (p, q) in enumerate(_EDGE_PAIRS) for c in _R3])
_PSUM = _mk((48, 16), [(3 * t + c, t) for t in range(16) for c in _R3])
_PD = _mk((16, 12), [(3 * a + c, 3 * i + c)
                     for i, a in enumerate(_DXN_ATOMS) for c in _R3])
_PN0 = _mk((32, 12), [(c, 3 * i + c) for i in range(4) for c in _R3])
_PQE = _mk((32, 36), [(12 + 3 * r + c, 9 * i + 3 * r + c)
                      for i in range(4) for r in _R3 for c in _R3])
_PDE = _mk((12, 36), [(3 * i + c, 9 * i + 3 * r + c)
                      for i in range(4) for r in _R3 for c in _R3])
_PSU = _mk((36, 16), [(9 * i + 3 * r + c, 3 * i + r)
                      for i in range(4) for r in _R3 for c in _R3])
_PNRM = _mk((16, 4), [(3 * i + c, i) for i in range(4) for c in _R3])
_PREP3 = _mk((4, 16), [(i, 3 * i + c) for i in range(4) for c in _R3])
_PREP16 = _mk((16, 256), [(t, 16 * t + k) for t in range(16) for k in range(16)])
_MUT16 = np.tile(_MU, 16).reshape(1, 256)

_NA = _mk((12, 18), [(3 * p + c, 3 * t + c)
                     for t, (p, q) in enumerate(_NODE_PAIRS) for c in _R3])
_NB = _mk((12, 18), [(3 * q + c, 3 * t + c)
                     for t, (p, q) in enumerate(_NODE_PAIRS) for c in _R3])
_NSUM = _mk((18, 6), [(3 * t + c, t) for t in range(6) for c in _R3])
_NREP = _mk((6, 96), [(t, 16 * t + k) for t in range(6) for k in range(16)])
_MUT6 = np.tile(_MU, 6).reshape(1, 96)
_DU0 = _mk((12, 3), [(3 + c, c) for c in _R3]) - _mk((12, 3), [(c, c) for c in _R3])
_DU1 = _mk((12, 3), [(6 + c, c) for c in _R3]) - _mk((12, 3), [(3 + c, c) for c in _R3])
_S1 = _mk((3, 3), [(1, 0), (2, 1), (0, 2)])
_S2 = _mk((3, 3), [(2, 0), (0, 1), (1, 2)])
_PB1 = _mk((3, 9), [(r, 3 * r) for r in _R3])
_PN9 = _mk((3, 9), [(r, 3 * r + 1) for r in _R3])
_PC9 = _mk((3, 9), [(r, 3 * r + 2) for r in _R3])
_E12T = _mk((12, 32), [(i, i) for i in range(12)])
_E9T = _mk((9, 32), [(j, 12 + j) for j in range(9)])
_E12X = _mk((12, 16), [(i, i) for i in range(12)])

_NODE_BLK = 1000
_EDGE_BLK = 1000

_CH = 128
_NROWS = _E // _CH
_NW = 32
_NJ = (_NROWS + _NW - 1) // _NW


_NODE_CONSTS = [_NA, _NB, _NSUM, _NREP, _MUT6, _DU0, _DU1, _S1, _S2,
                _PB1, _PN9, _PC9, _E12T, _E9T, _E12X]
_EDGE_CONSTS = [_PA, _PB, _PSUM, _PD, _PN0, _PQE, _PDE, _PSU, _PNRM,
                _PREP3, _PREP16, _MUT16]


def _full_spec(arr):
    return pl.BlockSpec(arr.shape, lambda i: (0, 0))


def _hidot(a, b):
    return lax.dot_general(a, b, (((1,), (0,)), ((), ())),
                           precision=lax.Precision.HIGHEST,
                           preferred_element_type=jnp.float32)


def _nrm3(v):
    n = jnp.sqrt(jnp.sum(v * v, axis=-1, keepdims=True))
    return v / jnp.where(n == 0.0, 1.0, n)


def _node_body(x_ref, wn_ref, bn_ref, g_ref, bb_ref,
               na_r, nb_r, nsum_r, nrep_r, mut6_r, du0_r, du1_r, s1_r, s2_r,
               pb1_r, pn9_r, pc9_r, e12t_r, e9t_r, e12x_r,
               hv_ref, ts_ref, tx_ref):
    s1 = s1_r[...]
    s2 = s2_r[...]

    def cross(a, b):
        return _hidot(a, s1) * _hidot(b, s2) - _hidot(a, s2) * _hidot(b, s1)

    X = x_ref[...]
    df = _hidot(X, na_r[...]) - _hidot(X, nb_r[...])
    d = jnp.sqrt(_hidot(df * df, nsum_r[...]))
    dr = _hidot(d, nrep_r[...])
    rb = jnp.exp(-(((dr - mut6_r[...]) * _INV_SIG) ** 2))
    z = jnp.dot(rb, wn_ref[...], preferred_element_type=jnp.float32) + bn_ref[...]
    mu = jnp.mean(z, axis=-1, keepdims=True)
    xc = z - mu
    sig = jnp.sqrt(jnp.sum(xc * xc, axis=-1, keepdims=True) * (1.0 / 127.0) + 1e-6)
    hv_ref[...] = g_ref[...] * xc / (sig + 1e-6) + bb_ref[...]

    u0 = _nrm3(_hidot(X, du0_r[...]))
    u1 = _nrm3(_hidot(X, du1_r[...]))
    n0 = _nrm3(cross(u0, u1))
    b1 = _nrm3(u0 - u1)
    cx = cross(b1, n0)
    q9 = (_hidot(b1, pb1_r[...]) + _hidot(n0, pn9_r[...])
          + _hidot(cx, pc9_r[...]))
    gid = _NODE_BLK * pl.program_id(0) + lax.broadcasted_iota(
        jnp.int32, (_NODE_BLK, 1), 0)
    q9 = jnp.where(gid < _N - 1, q9, 0.0)
    ts_ref[...] = _hidot(X, e12t_r[...]) + _hidot(q9, e9t_r[...])
    tx_ref[...] = _hidot(X, e12x_r[...])


def _node_call(x12, wn, bn, gn, bbn):
    f32 = jnp.float32
    return pl.pallas_call(
        _node_body,
        grid=(_N // _NODE_BLK,),
        in_specs=[
            pl.BlockSpec((_NODE_BLK, 12), lambda i: (i, 0)),
            pl.BlockSpec((96, 128), lambda i: (0, 0)),
            pl.BlockSpec((1, 128), lambda i: (0, 0)),
            pl.BlockSpec((1, 128), lambda i: (0, 0)),
            pl.BlockSpec((1, 128), lambda i: (0, 0)),
        ] + [_full_spec(c) for c in _NODE_CONSTS],
        out_specs=[
            pl.BlockSpec((_NODE_BLK, 128), lambda i: (i, 0)),
            pl.BlockSpec((_NODE_BLK, 32), lambda i: (i, 0)),
            pl.BlockSpec((_NODE_BLK, 16), lambda i: (i, 0)),
        ],
        out_shape=[
            jax.ShapeDtypeStruct((_N, 128), f32),
            jax.ShapeDtypeStruct((_N, 32), f32),
            jax.ShapeDtypeStruct((_N, 16), f32),
        ],
    )(x12, wn, bn, gn, bbn, *[jnp.asarray(c) for c in _NODE_CONSTS])


def _edge_body(s_ref, d_ref, wd_ref, wo_ref, be_ref, g_ref, bb_ref,
               pa_r, pb_r, psum_r, pd_r, pn0_r, pqe_r, pde_r, psu_r,
               pnrm_r, prep3_r, prep16_r, mut16_r, o_ref):
    S = s_ref[...]
    Dt = d_ref[...]
    df = _hidot(S, pa_r[...]) - _hidot(Dt, pb_r[...])
    dist = jnp.sqrt(_hidot(df * df, psum_r[...]) + 1e-6)
    dr = _hidot(dist, prep16_r[...])
    rb = jnp.exp(-(((dr - mut16_r[...]) * _INV_SIG) ** 2))
    dxn = _hidot(Dt, pd_r[...]) - _hidot(S, pn0_r[...])
    du = _hidot(_hidot(S, pqe_r[...]) * _hidot(dxn, pde_r[...]), psu_r[...])
    n = jnp.sqrt(_hidot(du * du, pnrm_r[...]))
    rinv = 1.0 / jnp.where(n == 0.0, 1.0, n)
    edir = du * _hidot(rinv, prep3_r[...])
    z = (jnp.dot(rb, wd_ref[...], preferred_element_type=jnp.float32)
         + jnp.dot(edir, wo_ref[...], preferred_element_type=jnp.float32)
         + be_ref[...])
    mu = jnp.mean(z, axis=-1, keepdims=True)
    xc = z - mu
    sig = jnp.sqrt(jnp.sum(xc * xc, axis=-1, keepdims=True) * (1.0 / 255.0) + 1e-6)
    o_ref[...] = g_ref[...] * xc / (sig + 1e-6) + bb_ref[...]


def _edge_call(s_rows, d_rows, wd, wo, be, ge, bbe):
    return pl.pallas_call(
        _edge_body,
        grid=(_E // _EDGE_BLK,),
        in_specs=[
            pl.BlockSpec((_EDGE_BLK, 32), lambda i: (i, 0)),
            pl.BlockSpec((_EDGE_BLK, 16), lambda i: (i, 0)),
            pl.BlockSpec((256, 256), lambda i: (0, 0)),
            pl.BlockSpec((16, 256), lambda i: (0, 0)),
            pl.BlockSpec((1, 256), lambda i: (0, 0)),
            pl.BlockSpec((1, 256), lambda i: (0, 0)),
            pl.BlockSpec((1, 256), lambda i: (0, 0)),
        ] + [_full_spec(c) for c in _EDGE_CONSTS],
        out_specs=pl.BlockSpec((_EDGE_BLK, 256), lambda i: (i, 0)),
        out_shape=jax.ShapeDtypeStruct((_E, 256), jnp.float32),
    )(s_rows, d_rows, wd, wo, be, ge, bbe,
      *[jnp.asarray(c) for c in _EDGE_CONSTS])


def _sc_gather(ts, tx, ii_s, ii_d, ie_s, ie_d):
    f32 = jnp.float32
    mesh = plsc.VectorSubcoreMesh(core_axis_name="c", subcore_axis_name="s")
    out_type = [
        jax.ShapeDtypeStruct((_E, 32), f32),
        jax.ShapeDtypeStruct((_E, 16), f32),
        jax.ShapeDtypeStruct((_E, 32), f32),
        jax.ShapeDtypeStruct((_E, 16), f32),
    ]

    @functools.partial(
        pl.kernel,
        out_type=out_type,
        mesh=mesh,
        compiler_params=pltpu.CompilerParams(use_tc_tiling_on_sc=False),
        scratch_types=[
            pltpu.VMEM((_CH,), jnp.int32),
            pltpu.VMEM((_CH, 32), f32),
            pltpu.VMEM((_CH, 16), f32),
            pltpu.SemaphoreType.DMA,
        ],
    )
    def k(ts_h, tx_h, iis_h, iid_h, ies_h, ied_h,
          o_is, o_id, o_es, o_ed, idx_v, row_s, row_d, sem):
        wid = lax.axis_index("s") * 2 + lax.axis_index("c")
        nj = jnp.maximum(0, jnp.minimum(_NJ, _NROWS - wid * _NJ))

        def body(j, carry):
            base = (wid * _NJ + j) * _CH
            pltpu.sync_copy(iis_h.at[pl.ds(base, _CH)], idx_v)
            pltpu.async_copy(ts_h.at[idx_v], row_s, sem).wait()
            pltpu.sync_copy(row_s, o_is.at[pl.ds(base, _CH)])
            pltpu.sync_copy(iid_h.at[pl.ds(base, _CH)], idx_v)
            pltpu.async_copy(tx_h.at[idx_v], row_d, sem).wait()
            pltpu.sync_copy(row_d, o_id.at[pl.ds(base, _CH)])
            pltpu.sync_copy(ies_h.at[pl.ds(base, _CH)], idx_v)
            pltpu.async_copy(ts_h.at[idx_v], row_s, sem).wait()
            pltpu.sync_copy(row_s, o_es.at[pl.ds(base, _CH)])
            pltpu.sync_copy(ied_h.at[pl.ds(base, _CH)], idx_v)
            pltpu.async_copy(tx_h.at[idx_v], row_d, sem).wait()
            pltpu.sync_copy(row_d, o_ed.at[pl.ds(base, _CH)])
            return carry

        lax.fori_loop(0, nj, body, 0)

    return k(ts, tx, ii_s, ii_d, ie_s, ie_d)


def kernel(X, E_in_idx, E_ex_idx, W_node, b_node, W_edge, b_edge,
           gain_nodes, bias_nodes, gain_edges, bias_edges):
    f32 = jnp.float32
    x12 = X.reshape(_N, 12)
    h_V, ts, tx = _node_call(x12, W_node, b_node.reshape(1, -1),
                             gain_nodes.reshape(1, -1), bias_nodes.reshape(1, -1))
    g_is, g_id, g_es, g_ed = _sc_gather(
        ts, tx, E_in_idx[0], E_in_idx[1], E_ex_idx[0], E_ex_idx[1])
    wd = W_edge[:256]
    wo = jnp.concatenate([W_edge[256:], jnp.zeros((4, 256), f32)], axis=0)
    be = b_edge.reshape(1, -1)
    ge = gain_edges.reshape(1, -1)
    bbe = bias_edges.reshape(1, -1)
    h_E_in = _edge_call(g_is, g_id, wd, wo, be, ge, bbe)
    h_E_ex = _edge_call(g_es, g_ed, wd, wo, be, ge, bbe)
    return h_V, h_E_in, h_E_ex


# speedup vs baseline: 5.3225x; 2.0961x over previous
import functools

import numpy as np
import jax
import jax.numpy as jnp
from jax import lax
from jax.experimental import pallas as pl
from jax.experimental.pallas import tpu as pltpu
from jax.experimental.pallas import tpu_sc as plsc

_N = 10000
_E = 160000
_R = 16
_MU = np.linspace(0.0, 20.0, _R).astype(np.float32)
_INV_SIG = np.float32(_R / 20.0)

_NODE_PAIRS = [(1, 0), (1, 2), (1, 3), (0, 2), (0, 3), (3, 2)]
_EDGE_PAIRS = [(1, 1), (1, 2), (2, 1), (1, 0), (0, 1), (1, 3), (3, 1),
               (2, 2), (2, 0), (0, 2), (2, 3), (3, 2), (0, 0), (0, 3),
               (3, 0), (3, 3)]
_DXN_ATOMS = [1, 0, 2, 3]


def _mk(shape, entries):
    m = np.zeros(shape, np.float32)
    for r, c in entries:
        m[r, c] = 1.0
    return m


_R3 = range(3)
_PA = _mk((32, 48), [(3 * p + c, 3 * t + c)
                     for t, (p, q) in enumerate(_EDGE_PAIRS) for c in _R3])
_PB = _mk((16, 48), [(3 * q + c, 3 * t + c)
                     for t, (p, q) in enumerate(_EDGE_PAIRS) for c in _R3])
_PSUM = _mk((48, 16), [(3 * t + c, t) for t in range(16) for c in _R3])
_PDXSEL = -_mk((48, 12), [(3 * t + c, 3 * i + c)
                          for i, t in enumerate([4, 12, 9, 13]) for c in _R3])
_PQE = _mk((32, 36), [(12 + 3 * r + c, 9 * i + 3 * r + c)
                      for i in range(4) for r in _R3 for c in _R3])
_PDE = _mk((12, 36), [(3 * i + c, 9 * i + 3 * r + c)
                      for i in range(4) for r in _R3 for c in _R3])
_PSU = _mk((36, 16), [(9 * i + 3 * r + c, 3 * i + r)
                      for i in range(4) for r in _R3 for c in _R3])
_PNRM = _mk((16, 4), [(3 * i + c, i) for i in range(4) for c in _R3])
_PREP3 = _mk((4, 16), [(i, 3 * i + c) for i in range(4) for c in _R3])
_PREP16 = _mk((16, 256), [(t, 16 * t + k) for t in range(16) for k in range(16)])
_MUT16 = np.tile(_MU, 16).reshape(1, 256)

_NA = _mk((12, 18), [(3 * p + c, 3 * t + c)
                     for t, (p, q) in enumerate(_NODE_PAIRS) for c in _R3])
_NB = _mk((12, 18), [(3 * q + c, 3 * t + c)
                     for t, (p, q) in enumerate(_NODE_PAIRS) for c in _R3])
_NSUM = _mk((18, 6), [(3 * t + c, t) for t in range(6) for c in _R3])
_NREP = _mk((6, 96), [(t, 16 * t + k) for t in range(6) for k in range(16)])
_MUT6 = np.tile(_MU, 6).reshape(1, 96)
_DU0 = _mk((12, 3), [(3 + c, c) for c in _R3]) - _mk((12, 3), [(c, c) for c in _R3])
_DU1 = _mk((12, 3), [(6 + c, c) for c in _R3]) - _mk((12, 3), [(3 + c, c) for c in _R3])
_S1 = _mk((3, 3), [(1, 0), (2, 1), (0, 2)])
_S2 = _mk((3, 3), [(2, 0), (0, 1), (1, 2)])
_PB1 = _mk((3, 9), [(r, 3 * r) for r in _R3])
_PN9 = _mk((3, 9), [(r, 3 * r + 1) for r in _R3])
_PC9 = _mk((3, 9), [(r, 3 * r + 2) for r in _R3])
_E12T = _mk((12, 32), [(i, i) for i in range(12)])
_E9T = _mk((9, 32), [(j, 12 + j) for j in range(9)])
_E12X = _mk((12, 16), [(i, i) for i in range(12)])

_NODE_BLK = 1000
_EDGE_BLK = 2000

_CH = 128
_NROWS = _E // _CH
_NW = 32
_NJ = (_NROWS + _NW - 1) // _NW


_NODE_CONSTS = [_NA, _NB, _NSUM, _NREP, _MUT6, _DU0, _DU1, _S1, _S2,
                _PB1, _PN9, _PC9, _E12T, _E9T, _E12X]
_EDGE_CONSTS = [_PA, _PB, _PSUM, _PDXSEL, _PQE, _PDE, _PSU, _PNRM,
                _PREP3, _MUT16]


def _full_spec(arr):
    return pl.BlockSpec(arr.shape, lambda i: (0, 0))


def _hidot(a, b):
    return lax.dot_general(a, b, (((1,), (0,)), ((), ())),
                           precision=lax.Precision.HIGHEST,
                           preferred_element_type=jnp.float32)


def _dot(a, b):
    return jnp.dot(a, b, preferred_element_type=jnp.float32)


def _splitdot(a, b):
    hi = a.astype(jnp.bfloat16).astype(jnp.float32)
    return _dot(hi, b) + _dot(a - hi, b)


def _nrm3(v):
    n = jnp.sqrt(jnp.sum(v * v, axis=-1, keepdims=True))
    return v / jnp.where(n == 0.0, 1.0, n)


def _node_body(x_ref, wn_ref, bn_ref, g_ref, bb_ref,
               na_r, nb_r, nsum_r, nrep_r, mut6_r, du0_r, du1_r, s1_r, s2_r,
               pb1_r, pn9_r, pc9_r, e12t_r, e9t_r, e12x_r,
               hv_ref, ts_ref, tx_ref):
    s1 = s1_r[...]
    s2 = s2_r[...]

    def cross(a, b):
        return _hidot(a, s1) * _hidot(b, s2) - _hidot(a, s2) * _hidot(b, s1)

    X = x_ref[...]
    df = _hidot(X, na_r[...]) - _hidot(X, nb_r[...])
    d = jnp.sqrt(_hidot(df * df, nsum_r[...]))
    dr = _hidot(d, nrep_r[...])
    rb = jnp.exp(-(((dr - mut6_r[...]) * _INV_SIG) ** 2))
    z = jnp.dot(rb, wn_ref[...], preferred_element_type=jnp.float32) + bn_ref[...]
    mu = jnp.mean(z, axis=-1, keepdims=True)
    xc = z - mu
    sig = jnp.sqrt(jnp.sum(xc * xc, axis=-1, keepdims=True) * (1.0 / 127.0) + 1e-6)
    hv_ref[...] = g_ref[...] * xc / (sig + 1e-6) + bb_ref[...]

    u0 = _nrm3(_hidot(X, du0_r[...]))
    u1 = _nrm3(_hidot(X, du1_r[...]))
    n0 = _nrm3(cross(u0, u1))
    b1 = _nrm3(u0 - u1)
    cx = cross(b1, n0)
    q9 = (_hidot(b1, pb1_r[...]) + _hidot(n0, pn9_r[...])
          + _hidot(cx, pc9_r[...]))
    gid = _NODE_BLK * pl.program_id(0) + lax.broadcasted_iota(
        jnp.int32, (_NODE_BLK, 1), 0)
    q9 = jnp.where(gid < _N - 1, q9, 0.0)
    ts_ref[...] = _hidot(X, e12t_r[...]) + _hidot(q9, e9t_r[...])
    tx_ref[...] = _hidot(X, e12x_r[...])


def _node_call(x12, wn, bn, gn, bbn):
    f32 = jnp.float32
    return pl.pallas_call(
        _node_body,
        grid=(_N // _NODE_BLK,),
        in_specs=[
            pl.BlockSpec((_NODE_BLK, 12), lambda i: (i, 0)),
            pl.BlockSpec((96, 128), lambda i: (0, 0)),
            pl.BlockSpec((1, 128), lambda i: (0, 0)),
            pl.BlockSpec((1, 128), lambda i: (0, 0)),
            pl.BlockSpec((1, 128), lambda i: (0, 0)),
        ] + [_full_spec(c) for c in _NODE_CONSTS],
        out_specs=[
            pl.BlockSpec((_NODE_BLK, 128), lambda i: (i, 0)),
            pl.BlockSpec((_NODE_BLK, 32), lambda i: (i, 0)),
            pl.BlockSpec((_NODE_BLK, 16), lambda i: (i, 0)),
        ],
        out_shape=[
            jax.ShapeDtypeStruct((_N, 128), f32),
            jax.ShapeDtypeStruct((_N, 32), f32),
            jax.ShapeDtypeStruct((_N, 16), f32),
        ],
    )(x12, wn, bn, gn, bbn, *[jnp.asarray(c) for c in _NODE_CONSTS])


def _edge_body(s_ref, d_ref, wd_ref, wo_ref, be_ref, g_ref, bb_ref,
               pa_r, pb_r, psum_r, pdxsel_r, pqe_r, pde_r, psu_r,
               pnrm_r, prep3_r, mut16_r, o_ref):
    S = s_ref[...]
    Dt = d_ref[...]
    df = _splitdot(S, pa_r[...]) - _splitdot(Dt, pb_r[...])
    dist = jnp.sqrt(_splitdot(df * df, psum_r[...]) + 1e-6)
    dr = jnp.broadcast_to(dist[:, :, None], (_EDGE_BLK, 16, 16)).reshape(
        _EDGE_BLK, 256)
    rb = jnp.exp(-(((dr - mut16_r[...]) * _INV_SIG) ** 2))
    dxn = _dot(df, pdxsel_r[...])
    du = _dot(_dot(S, pqe_r[...]) * _dot(dxn, pde_r[...]), psu_r[...])
    n = jnp.sqrt(_dot(du * du, pnrm_r[...]))
    rinv = 1.0 / jnp.where(n == 0.0, 1.0, n)
    edir = du * _dot(rinv, prep3_r[...])
    z = (_dot(rb, wd_ref[...]) + _dot(edir, wo_ref[...]) + be_ref[...])
    mu = jnp.mean(z, axis=-1, keepdims=True)
    xc = z - mu
    sig = jnp.sqrt(jnp.sum(xc * xc, axis=-1, keepdims=True) * (1.0 / 255.0) + 1e-6)
    o_ref[...] = g_ref[...] * xc / (sig + 1e-6) + bb_ref[...]


def _edge_call(s_rows, d_rows, wd, wo, be, ge, bbe):
    return pl.pallas_call(
        _edge_body,
        grid=(_E // _EDGE_BLK,),
        in_specs=[
            pl.BlockSpec((_EDGE_BLK, 32), lambda i: (i, 0)),
            pl.BlockSpec((_EDGE_BLK, 16), lambda i: (i, 0)),
            pl.BlockSpec((256, 256), lambda i: (0, 0)),
            pl.BlockSpec((16, 256), lambda i: (0, 0)),
            pl.BlockSpec((1, 256), lambda i: (0, 0)),
            pl.BlockSpec((1, 256), lambda i: (0, 0)),
            pl.BlockSpec((1, 256), lambda i: (0, 0)),
        ] + [_full_spec(c) for c in _EDGE_CONSTS],
        out_specs=pl.BlockSpec((_EDGE_BLK, 256), lambda i: (i, 0)),
        out_shape=jax.ShapeDtypeStruct((_E, 256), jnp.float32),
    )(s_rows, d_rows, wd, wo, be, ge, bbe,
      *[jnp.asarray(c) for c in _EDGE_CONSTS])


def _sc_gather(ts, tx, ii_s, ii_d, ie_s, ie_d):
    f32 = jnp.float32
    mesh = plsc.VectorSubcoreMesh(core_axis_name="c", subcore_axis_name="s")
    out_type = [
        jax.ShapeDtypeStruct((_E, 32), f32),
        jax.ShapeDtypeStruct((_E, 16), f32),
        jax.ShapeDtypeStruct((_E, 32), f32),
        jax.ShapeDtypeStruct((_E, 16), f32),
    ]

    @functools.partial(
        pl.kernel,
        out_type=out_type,
        mesh=mesh,
        compiler_params=pltpu.CompilerParams(use_tc_tiling_on_sc=False),
        scratch_types=[
            pltpu.VMEM((_CH,), jnp.int32),
            pltpu.VMEM((_CH, 32), f32),
            pltpu.VMEM((_CH, 16), f32),
            pltpu.SemaphoreType.DMA,
        ],
    )
    def k(ts_h, tx_h, iis_h, iid_h, ies_h, ied_h,
          o_is, o_id, o_es, o_ed, idx_v, row_s, row_d, sem):
        wid = lax.axis_index("s") * 2 + lax.axis_index("c")
        nj = jnp.maximum(0, jnp.minimum(_NJ, _NROWS - wid * _NJ))

        def body(j, carry):
            base = (wid * _NJ + j) * _CH
            pltpu.sync_copy(iis_h.at[pl.ds(base, _CH)], idx_v)
            pltpu.async_copy(ts_h.at[idx_v], row_s, sem).wait()
            pltpu.sync_copy(row_s, o_is.at[pl.ds(base, _CH)])
            pltpu.sync_copy(iid_h.at[pl.ds(base, _CH)], idx_v)
            pltpu.async_copy(tx_h.at[idx_v], row_d, sem).wait()
            pltpu.sync_copy(row_d, o_id.at[pl.ds(base, _CH)])
            pltpu.sync_copy(ies_h.at[pl.ds(base, _CH)], idx_v)
            pltpu.async_copy(ts_h.at[idx_v], row_s, sem).wait()
            pltpu.sync_copy(row_s, o_es.at[pl.ds(base, _CH)])
            pltpu.sync_copy(ied_h.at[pl.ds(base, _CH)], idx_v)
            pltpu.async_copy(tx_h.at[idx_v], row_d, sem).wait()
            pltpu.sync_copy(row_d, o_ed.at[pl.ds(base, _CH)])
            return carry

        lax.fori_loop(0, nj, body, 0)

    return k(ts, tx, ii_s, ii_d, ie_s, ie_d)


def kernel(X, E_in_idx, E_ex_idx, W_node, b_node, W_edge, b_edge,
           gain_nodes, bias_nodes, gain_edges, bias_edges):
    f32 = jnp.float32
    x12 = X.reshape(_N, 12)
    h_V, ts, tx = _node_call(x12, W_node, b_node.reshape(1, -1),
                             gain_nodes.reshape(1, -1), bias_nodes.reshape(1, -1))
    g_is, g_id, g_es, g_ed = _sc_gather(
        ts, tx, E_in_idx[0], E_in_idx[1], E_ex_idx[0], E_ex_idx[1])
    wd = W_edge[:256]
    wo = jnp.concatenate([W_edge[256:], jnp.zeros((4, 256), f32)], axis=0)
    be = b_edge.reshape(1, -1)
    ge = gain_edges.reshape(1, -1)
    bbe = bias_edges.reshape(1, -1)
    h_E_in = _edge_call(g_is, g_id, wd, wo, be, ge, bbe)
    h_E_ex = _edge_call(g_es, g_ed, wd, wo, be, ge, bbe)
    return h_V, h_E_in, h_E_ex


# speedup vs baseline: 11.2401x; 2.1118x over previous
import functools

import numpy as np
import jax
import jax.numpy as jnp
from jax import lax
from jax.experimental import pallas as pl
from jax.experimental.pallas import tpu as pltpu
from jax.experimental.pallas import tpu_sc as plsc

_N = 10000
_E = 160000
_R = 16
_MU = np.linspace(0.0, 20.0, _R).astype(np.float32)
_INV_SIG = np.float32(_R / 20.0)

_NODE_PAIRS = [(1, 0), (1, 2), (1, 3), (0, 2), (0, 3), (3, 2)]
_EDGE_PAIRS = [(1, 1), (1, 2), (2, 1), (1, 0), (0, 1), (1, 3), (3, 1),
               (2, 2), (2, 0), (0, 2), (2, 3), (3, 2), (0, 0), (0, 3),
               (3, 0), (3, 3)]
_DXN_ATOMS = [1, 0, 2, 3]


def _mk(shape, entries):
    m = np.zeros(shape, np.float32)
    for r, c in entries:
        m[r, c] = 1.0
    return m


_R3 = range(3)
_PA = _mk((32, 48), [(3 * p + c, 3 * t + c)
                     for t, (p, q) in enumerate(_EDGE_PAIRS) for c in _R3])
_PB = _mk((16, 48), [(3 * q + c, 3 * t + c)
                     for t, (p, q) in enumerate(_EDGE_PAIRS) for c in _R3])
_PSUM = _mk((48, 16), [(3 * t + c, t) for t in range(16) for c in _R3])
_PDXSEL = -_mk((48, 12), [(3 * t + c, 3 * i + c)
                          for i, t in enumerate([4, 12, 9, 13]) for c in _R3])
_PQE = _mk((32, 36), [(12 + 3 * r + c, 9 * i + 3 * r + c)
                      for i in range(4) for r in _R3 for c in _R3])
_PDE = _mk((12, 36), [(3 * i + c, 9 * i + 3 * r + c)
                      for i in range(4) for r in _R3 for c in _R3])
_PSU = _mk((36, 16), [(9 * i + 3 * r + c, 3 * i + r)
                      for i in range(4) for r in _R3 for c in _R3])
_PNRM = _mk((16, 4), [(3 * i + c, i) for i in range(4) for c in _R3])
_PREP3 = _mk((4, 16), [(i, 3 * i + c) for i in range(4) for c in _R3])
_PREP16 = _mk((16, 256), [(t, 16 * t + k) for t in range(16) for k in range(16)])
_MUT16 = np.tile(_MU, 16).reshape(1, 256)

_NA = _mk((12, 18), [(3 * p + c, 3 * t + c)
                     for t, (p, q) in enumerate(_NODE_PAIRS) for c in _R3])
_NB = _mk((12, 18), [(3 * q + c, 3 * t + c)
                     for t, (p, q) in enumerate(_NODE_PAIRS) for c in _R3])
_NSUM = _mk((18, 6), [(3 * t + c, t) for t in range(6) for c in _R3])
_NREP = _mk((6, 96), [(t, 16 * t + k) for t in range(6) for k in range(16)])
_MUT6 = np.tile(_MU, 6).reshape(1, 96)
_DU0 = _mk((12, 3), [(3 + c, c) for c in _R3]) - _mk((12, 3), [(c, c) for c in _R3])
_DU1 = _mk((12, 3), [(6 + c, c) for c in _R3]) - _mk((12, 3), [(3 + c, c) for c in _R3])
_S1 = _mk((3, 3), [(1, 0), (2, 1), (0, 2)])
_S2 = _mk((3, 3), [(2, 0), (0, 1), (1, 2)])
_PB1 = _mk((3, 9), [(r, 3 * r) for r in _R3])
_PN9 = _mk((3, 9), [(r, 3 * r + 1) for r in _R3])
_PC9 = _mk((3, 9), [(r, 3 * r + 2) for r in _R3])
_E12T = _mk((12, 32), [(i, i) for i in range(12)])
_E9T = _mk((9, 32), [(j, 12 + j) for j in range(9)])
_E12X = _mk((12, 16), [(i, i) for i in range(12)])

_NODE_BLK = 2000
_EDGE_BLK = 2000

_CH = 128
_NROWS = _E // _CH
_NW = 32
_NJ = (_NROWS + _NW - 1) // _NW


_NODE_CONSTS = [_NA, _NB, _NSUM, _NREP, _MUT6, _DU0, _DU1, _S1, _S2,
                _PB1, _PN9, _PC9, _E12T, _E9T, _E12X]
_EDGE_CONSTS = [_PA, _PB, _PSUM, _PDXSEL, _PQE, _PDE, _PSU, _PNRM,
                _PREP3, _PREP16, _MUT16]


def _full_spec(arr):
    return pl.BlockSpec(arr.shape, lambda i: (0, 0))


def _hidot(a, b):
    return lax.dot_general(a, b, (((1,), (0,)), ((), ())),
                           precision=lax.Precision.HIGHEST,
                           preferred_element_type=jnp.float32)


def _dot(a, b):
    return jnp.dot(a, b, preferred_element_type=jnp.float32)


def _splitdot(a, b):
    hi = a.astype(jnp.bfloat16).astype(jnp.float32)
    return _dot(hi, b) + _dot(a - hi, b)


def _nrm3(v):
    n = jnp.sqrt(jnp.sum(v * v, axis=-1, keepdims=True))
    return v / jnp.where(n == 0.0, 1.0, n)


def _node_body(x_ref, wn_ref, bn_ref, g_ref, bb_ref,
               na_r, nb_r, nsum_r, nrep_r, mut6_r, du0_r, du1_r, s1_r, s2_r,
               pb1_r, pn9_r, pc9_r, e12t_r, e9t_r, e12x_r,
               hv_ref, ts_ref, tx_ref):
    s1 = s1_r[...]
    s2 = s2_r[...]

    def cross(a, b):
        return _dot(a, s1) * _dot(b, s2) - _dot(a, s2) * _dot(b, s1)

    X = x_ref[...]
    df = _splitdot(X, na_r[...]) - _splitdot(X, nb_r[...])
    d = jnp.sqrt(_splitdot(df * df, nsum_r[...]))
    dr = _splitdot(d, nrep_r[...])
    rb = jnp.exp(-(((dr - mut6_r[...]) * _INV_SIG) ** 2))
    z = jnp.dot(rb, wn_ref[...], preferred_element_type=jnp.float32) + bn_ref[...]
    mu = jnp.mean(z, axis=-1, keepdims=True)
    xc = z - mu
    sig = jnp.sqrt(jnp.sum(xc * xc, axis=-1, keepdims=True) * (1.0 / 127.0) + 1e-6)
    hv_ref[...] = g_ref[...] * xc / (sig + 1e-6) + bb_ref[...]

    u0 = _nrm3(_dot(X, du0_r[...]))
    u1 = _nrm3(_dot(X, du1_r[...]))
    n0 = _nrm3(cross(u0, u1))
    b1 = _nrm3(u0 - u1)
    cx = cross(b1, n0)
    q9 = (_dot(b1, pb1_r[...]) + _dot(n0, pn9_r[...])
          + _dot(cx, pc9_r[...]))
    gid = _NODE_BLK * pl.program_id(0) + lax.broadcasted_iota(
        jnp.int32, (_NODE_BLK, 1), 0)
    q9 = jnp.where(gid < _N - 1, q9, 0.0)
    ts_ref[...] = _splitdot(X, e12t_r[...]) + _dot(q9, e9t_r[...])
    tx_ref[...] = _splitdot(X, e12x_r[...])


def _node_call(x12, wn, bn, gn, bbn):
    f32 = jnp.float32
    return pl.pallas_call(
        _node_body,
        grid=(_N // _NODE_BLK,),
        in_specs=[
            pl.BlockSpec((_NODE_BLK, 12), lambda i: (i, 0)),
            pl.BlockSpec((96, 128), lambda i: (0, 0)),
            pl.BlockSpec((1, 128), lambda i: (0, 0)),
            pl.BlockSpec((1, 128), lambda i: (0, 0)),
            pl.BlockSpec((1, 128), lambda i: (0, 0)),
        ] + [_full_spec(c) for c in _NODE_CONSTS],
        out_specs=[
            pl.BlockSpec((_NODE_BLK, 128), lambda i: (i, 0)),
            pl.BlockSpec((_NODE_BLK, 32), lambda i: (i, 0)),
            pl.BlockSpec((_NODE_BLK, 16), lambda i: (i, 0)),
        ],
        out_shape=[
            jax.ShapeDtypeStruct((_N, 128), f32),
            jax.ShapeDtypeStruct((_N, 32), f32),
            jax.ShapeDtypeStruct((_N, 16), f32),
        ],
    )(x12, wn, bn, gn, bbn, *[jnp.asarray(c) for c in _NODE_CONSTS])


def _edge_body(s_ref, d_ref, wd_ref, wo_ref, be_ref, g_ref, bb_ref,
               pa_r, pb_r, psum_r, pdxsel_r, pqe_r, pde_r, psu_r,
               pnrm_r, prep3_r, prep16_r, mut16_r, o_ref):
    S = s_ref[...]
    Dt = d_ref[...]
    df = _splitdot(S, pa_r[...]) - _splitdot(Dt, pb_r[...])
    dist = jnp.sqrt(_splitdot(df * df, psum_r[...]) + 1e-6)
    dr = _splitdot(dist, prep16_r[...])
    rb = jnp.exp(-(((dr - mut16_r[...]) * _INV_SIG) ** 2))
    dxn = _dot(df, pdxsel_r[...])
    du = _dot(_dot(S, pqe_r[...]) * _dot(dxn, pde_r[...]), psu_r[...])
    n = jnp.sqrt(_dot(du * du, pnrm_r[...]))
    rinv = 1.0 / jnp.where(n == 0.0, 1.0, n)
    edir = du * _dot(rinv, prep3_r[...])
    z = (_dot(rb, wd_ref[...]) + _dot(edir, wo_ref[...]) + be_ref[...])
    mu = jnp.mean(z, axis=-1, keepdims=True)
    xc = z - mu
    sig = jnp.sqrt(jnp.sum(xc * xc, axis=-1, keepdims=True) * (1.0 / 255.0) + 1e-6)
    o_ref[...] = g_ref[...] * xc / (sig + 1e-6) + bb_ref[...]


def _edge_call(s_rows, d_rows, wd, wo, be, ge, bbe):
    return pl.pallas_call(
        _edge_body,
        grid=(_E // _EDGE_BLK,),
        in_specs=[
            pl.BlockSpec((_EDGE_BLK, 32), lambda i: (i, 0)),
            pl.BlockSpec((_EDGE_BLK, 16), lambda i: (i, 0)),
            pl.BlockSpec((256, 256), lambda i: (0, 0)),
            pl.BlockSpec((16, 256), lambda i: (0, 0)),
            pl.BlockSpec((1, 256), lambda i: (0, 0)),
            pl.BlockSpec((1, 256), lambda i: (0, 0)),
            pl.BlockSpec((1, 256), lambda i: (0, 0)),
        ] + [_full_spec(c) for c in _EDGE_CONSTS],
        out_specs=pl.BlockSpec((_EDGE_BLK, 256), lambda i: (i, 0)),
        out_shape=jax.ShapeDtypeStruct((_E, 256), jnp.float32),
    )(s_rows, d_rows, wd, wo, be, ge, bbe,
      *[jnp.asarray(c) for c in _EDGE_CONSTS])


def _sc_gather(ts, tx, ii_s, ii_d, ie_s, ie_d):
    f32 = jnp.float32
    mesh = plsc.VectorSubcoreMesh(core_axis_name="c", subcore_axis_name="s")
    out_type = [
        jax.ShapeDtypeStruct((_E, 32), f32),
        jax.ShapeDtypeStruct((_E, 16), f32),
        jax.ShapeDtypeStruct((_E, 32), f32),
        jax.ShapeDtypeStruct((_E, 16), f32),
    ]

    @functools.partial(
        pl.kernel,
        out_type=out_type,
        mesh=mesh,
        compiler_params=pltpu.CompilerParams(use_tc_tiling_on_sc=False),
        scratch_types=[
            pltpu.VMEM((_CH,), jnp.int32),
            pltpu.VMEM((_CH, 32), f32),
            pltpu.VMEM((_CH, 16), f32),
            pltpu.SemaphoreType.DMA,
        ],
    )
    def k(ts_h, tx_h, iis_h, iid_h, ies_h, ied_h,
          o_is, o_id, o_es, o_ed, idx_v, row_s, row_d, sem):
        wid = lax.axis_index("s") * 2 + lax.axis_index("c")
        nj = jnp.maximum(0, jnp.minimum(_NJ, _NROWS - wid * _NJ))

        def body(j, carry):
            base = (wid * _NJ + j) * _CH
            pltpu.sync_copy(iis_h.at[pl.ds(base, _CH)], idx_v)
            pltpu.async_copy(ts_h.at[idx_v], row_s, sem).wait()
            pltpu.sync_copy(row_s, o_is.at[pl.ds(base, _CH)])
            pltpu.sync_copy(iid_h.at[pl.ds(base, _CH)], idx_v)
            pltpu.async_copy(tx_h.at[idx_v], row_d, sem).wait()
            pltpu.sync_copy(row_d, o_id.at[pl.ds(base, _CH)])
            pltpu.sync_copy(ies_h.at[pl.ds(base, _CH)], idx_v)
            pltpu.async_copy(ts_h.at[idx_v], row_s, sem).wait()
            pltpu.sync_copy(row_s, o_es.at[pl.ds(base, _CH)])
            pltpu.sync_copy(ied_h.at[pl.ds(base, _CH)], idx_v)
            pltpu.async_copy(tx_h.at[idx_v], row_d, sem).wait()
            pltpu.sync_copy(row_d, o_ed.at[pl.ds(base, _CH)])
            return carry

        lax.fori_loop(0, nj, body, 0)

    return k(ts, tx, ii_s, ii_d, ie_s, ie_d)


def kernel(X, E_in_idx, E_ex_idx, W_node, b_node, W_edge, b_edge,
           gain_nodes, bias_nodes, gain_edges, bias_edges):
    f32 = jnp.float32
    x12 = X.reshape(_N, 12)
    h_V, ts, tx = _node_call(x12, W_node, b_node.reshape(1, -1),
                             gain_nodes.reshape(1, -1), bias_nodes.reshape(1, -1))
    g_is, g_id, g_es, g_ed = _sc_gather(
        ts, tx, E_in_idx[0], E_in_idx[1], E_ex_idx[0], E_ex_idx[1])
    wd = W_edge[:256]
    wo = jnp.concatenate([W_edge[256:], jnp.zeros((4, 256), f32)], axis=0)
    be = b_edge.reshape(1, -1)
    ge = gain_edges.reshape(1, -1)
    bbe = bias_edges.reshape(1, -1)
    h_E_in = _edge_call(g_is, g_id, wd, wo, be, ge, bbe)
    h_E_ex = _edge_call(g_es, g_ed, wd, wo, be, ge, bbe)
    return h_V, h_E_in, h_E_ex


# speedup vs baseline: 12.7974x; 1.1386x over previous
import functools

import numpy as np
import jax
import jax.numpy as jnp
from jax import lax
from jax.experimental import pallas as pl
from jax.experimental.pallas import tpu as pltpu
from jax.experimental.pallas import tpu_sc as plsc

_N = 10000
_E = 160000
_R = 16
_MU = np.linspace(0.0, 20.0, _R).astype(np.float32)
_INV_SIG = np.float32(_R / 20.0)

_NODE_PAIRS = [(1, 0), (1, 2), (1, 3), (0, 2), (0, 3), (3, 2)]
_EDGE_PAIRS = [(1, 1), (1, 2), (2, 1), (1, 0), (0, 1), (1, 3), (3, 1),
               (2, 2), (2, 0), (0, 2), (2, 3), (3, 2), (0, 0), (0, 3),
               (3, 0), (3, 3)]
_DXN_ATOMS = [1, 0, 2, 3]


def _mk(shape, entries):
    m = np.zeros(shape, np.float32)
    for r, c in entries:
        m[r, c] = 1.0
    return m


_R3 = range(3)
_PA = _mk((32, 48), [(3 * p + c, 3 * t + c)
                     for t, (p, q) in enumerate(_EDGE_PAIRS) for c in _R3])
_PB = _mk((16, 48), [(3 * q + c, 3 * t + c)
                     for t, (p, q) in enumerate(_EDGE_PAIRS) for c in _R3])
_PSUM = _mk((48, 16), [(3 * t + c, t) for t in range(16) for c in _R3])
_PDXSEL = -_mk((48, 12), [(3 * t + c, 3 * i + c)
                          for i, t in enumerate([4, 12, 9, 13]) for c in _R3])
_PQE = _mk((32, 36), [(12 + 3 * r + c, 9 * i + 3 * r + c)
                      for i in range(4) for r in _R3 for c in _R3])
_PDE = _mk((12, 36), [(3 * i + c, 9 * i + 3 * r + c)
                      for i in range(4) for r in _R3 for c in _R3])
_PSU = _mk((36, 16), [(9 * i + 3 * r + c, 3 * i + r)
                      for i in range(4) for r in _R3 for c in _R3])
_PNRM = _mk((16, 4), [(3 * i + c, i) for i in range(4) for c in _R3])
_PREP3 = _mk((4, 16), [(i, 3 * i + c) for i in range(4) for c in _R3])
_PREP16 = _mk((16, 256), [(t, 16 * t + k) for t in range(16) for k in range(16)])
_MUT16 = np.tile(_MU, 16).reshape(1, 256)

_NA = _mk((12, 18), [(3 * p + c, 3 * t + c)
                     for t, (p, q) in enumerate(_NODE_PAIRS) for c in _R3])
_NB = _mk((12, 18), [(3 * q + c, 3 * t + c)
                     for t, (p, q) in enumerate(_NODE_PAIRS) for c in _R3])
_NSUM = _mk((18, 6), [(3 * t + c, t) for t in range(6) for c in _R3])
_NREP = _mk((6, 96), [(t, 16 * t + k) for t in range(6) for k in range(16)])
_MUT6 = np.tile(_MU, 6).reshape(1, 96)
_DU0 = _mk((12, 3), [(3 + c, c) for c in _R3]) - _mk((12, 3), [(c, c) for c in _R3])
_DU1 = _mk((12, 3), [(6 + c, c) for c in _R3]) - _mk((12, 3), [(3 + c, c) for c in _R3])
_S1 = _mk((3, 3), [(1, 0), (2, 1), (0, 2)])
_S2 = _mk((3, 3), [(2, 0), (0, 1), (1, 2)])
_PB1 = _mk((3, 9), [(r, 3 * r) for r in _R3])
_PN9 = _mk((3, 9), [(r, 3 * r + 1) for r in _R3])
_PC9 = _mk((3, 9), [(r, 3 * r + 2) for r in _R3])
_E12T = _mk((12, 32), [(i, i) for i in range(12)])
_E9T = _mk((9, 32), [(j, 12 + j) for j in range(9)])
_E12X = _mk((12, 16), [(i, i) for i in range(12)])

_NODE_BLK = 2000
_EDGE_BLK = 2000

_CH = 128
_NROWS = _E // _CH
_NW = 32
_NJ = (_NROWS + _NW - 1) // _NW


_NODE_CONSTS = [_NA, _NB, _NSUM, _NREP, _MUT6, _DU0, _DU1, _S1, _S2,
                _PB1, _PN9, _PC9, _E12T, _E9T, _E12X]
_EDGE_CONSTS = [_PA, _PB, _PSUM, _PDXSEL, _PQE, _PDE, _PSU, _PNRM,
                _PREP3, _PREP16, _MUT16]


def _full_spec(arr):
    return pl.BlockSpec(arr.shape, lambda i: (0, 0))


def _hidot(a, b):
    return lax.dot_general(a, b, (((1,), (0,)), ((), ())),
                           precision=lax.Precision.HIGHEST,
                           preferred_element_type=jnp.float32)


def _dot(a, b):
    return jnp.dot(a, b, preferred_element_type=jnp.float32)


def _splitdot(a, b):
    hi = a.astype(jnp.bfloat16).astype(jnp.float32)
    return _dot(hi, b) + _dot(a - hi, b)


def _nrm3(v):
    n = jnp.sqrt(jnp.sum(v * v, axis=-1, keepdims=True))
    return v / jnp.where(n == 0.0, 1.0, n)


def _node_body(x_ref, wn_ref, bn_ref, g_ref, bb_ref,
               na_r, nb_r, nsum_r, nrep_r, mut6_r, du0_r, du1_r, s1_r, s2_r,
               pb1_r, pn9_r, pc9_r, e12t_r, e9t_r, e12x_r,
               hv_ref, ts_ref, tx_ref):
    s1 = s1_r[...]
    s2 = s2_r[...]

    def cross(a, b):
        return _dot(a, s1) * _dot(b, s2) - _dot(a, s2) * _dot(b, s1)

    X = x_ref[...]
    df = _splitdot(X, na_r[...]) - _splitdot(X, nb_r[...])
    d = jnp.sqrt(_splitdot(df * df, nsum_r[...]))
    dr = _splitdot(d, nrep_r[...])
    rb = jnp.exp(-(((dr - mut6_r[...]) * _INV_SIG) ** 2))
    z = jnp.dot(rb, wn_ref[...], preferred_element_type=jnp.float32) + bn_ref[...]
    mu = jnp.mean(z, axis=-1, keepdims=True)
    xc = z - mu
    sig = jnp.sqrt(jnp.sum(xc * xc, axis=-1, keepdims=True) * (1.0 / 127.0) + 1e-6)
    hv_ref[...] = g_ref[...] * xc / (sig + 1e-6) + bb_ref[...]

    u0 = _nrm3(_dot(X, du0_r[...]))
    u1 = _nrm3(_dot(X, du1_r[...]))
    n0 = _nrm3(cross(u0, u1))
    b1 = _nrm3(u0 - u1)
    cx = cross(b1, n0)
    q9 = (_dot(b1, pb1_r[...]) + _dot(n0, pn9_r[...])
          + _dot(cx, pc9_r[...]))
    gid = _NODE_BLK * pl.program_id(0) + lax.broadcasted_iota(
        jnp.int32, (_NODE_BLK, 1), 0)
    q9 = jnp.where(gid < _N - 1, q9, 0.0)
    ts_ref[...] = _splitdot(X, e12t_r[...]) + _dot(q9, e9t_r[...])
    tx_ref[...] = _splitdot(X, e12x_r[...])


def _node_call(x12, wn, bn, gn, bbn):
    f32 = jnp.float32
    return pl.pallas_call(
        _node_body,
        grid=(_N // _NODE_BLK,),
        in_specs=[
            pl.BlockSpec((_NODE_BLK, 12), lambda i: (i, 0)),
            pl.BlockSpec((96, 128), lambda i: (0, 0)),
            pl.BlockSpec((1, 128), lambda i: (0, 0)),
            pl.BlockSpec((1, 128), lambda i: (0, 0)),
            pl.BlockSpec((1, 128), lambda i: (0, 0)),
        ] + [_full_spec(c) for c in _NODE_CONSTS],
        out_specs=[
            pl.BlockSpec((_NODE_BLK, 128), lambda i: (i, 0)),
            pl.BlockSpec((_NODE_BLK, 32), lambda i: (i, 0)),
            pl.BlockSpec((_NODE_BLK, 16), lambda i: (i, 0)),
        ],
        out_shape=[
            jax.ShapeDtypeStruct((_N, 128), f32),
            jax.ShapeDtypeStruct((_N, 32), f32),
            jax.ShapeDtypeStruct((_N, 16), f32),
        ],
    )(x12, wn, bn, gn, bbn, *[jnp.asarray(c) for c in _NODE_CONSTS])


def _edge_body(s_ref, d_ref, wd_ref, wo_ref, be_ref, g_ref, bb_ref,
               pa_r, pb_r, psum_r, pdxsel_r, pqe_r, pde_r, psu_r,
               pnrm_r, prep3_r, prep16_r, mut16_r, o_ref):
    S = s_ref[...]
    Dt = d_ref[...]
    df = _splitdot(S, pa_r[...]) - _splitdot(Dt, pb_r[...])
    dist = jnp.sqrt(_splitdot(df * df, psum_r[...]) + 1e-6)
    dr = _splitdot(dist, prep16_r[...])
    rb = jnp.exp(-(((dr - mut16_r[...]) * _INV_SIG) ** 2))
    dxn = _dot(df, pdxsel_r[...])
    du = _dot(_dot(S, pqe_r[...]) * _dot(dxn, pde_r[...]), psu_r[...])
    n = jnp.sqrt(_dot(du * du, pnrm_r[...]))
    rinv = 1.0 / jnp.where(n == 0.0, 1.0, n)
    edir = du * _dot(rinv, prep3_r[...])
    z = (_dot(rb, wd_ref[...]) + _dot(edir, wo_ref[...]) + be_ref[...])
    mu = jnp.mean(z, axis=-1, keepdims=True)
    xc = z - mu
    sig = jnp.sqrt(jnp.sum(xc * xc, axis=-1, keepdims=True) * (1.0 / 255.0) + 1e-6)
    o_ref[...] = g_ref[...] * xc / (sig + 1e-6) + bb_ref[...]


def _edge_call(s_rows, d_rows, wd, wo, be, ge, bbe):
    return pl.pallas_call(
        _edge_body,
        grid=(_E // _EDGE_BLK,),
        in_specs=[
            pl.BlockSpec((_EDGE_BLK, 32), lambda i: (i, 0)),
            pl.BlockSpec((_EDGE_BLK, 16), lambda i: (i, 0)),
            pl.BlockSpec((256, 256), lambda i: (0, 0)),
            pl.BlockSpec((16, 256), lambda i: (0, 0)),
            pl.BlockSpec((1, 256), lambda i: (0, 0)),
            pl.BlockSpec((1, 256), lambda i: (0, 0)),
            pl.BlockSpec((1, 256), lambda i: (0, 0)),
        ] + [_full_spec(c) for c in _EDGE_CONSTS],
        out_specs=pl.BlockSpec((_EDGE_BLK, 256), lambda i: (i, 0)),
        out_shape=jax.ShapeDtypeStruct((_E, 256), jnp.float32),
    )(s_rows, d_rows, wd, wo, be, ge, bbe,
      *[jnp.asarray(c) for c in _EDGE_CONSTS])


def _sc_gather(ts, tx, ii_s, ii_d, ie_s, ie_d):
    f32 = jnp.float32
    mesh = plsc.VectorSubcoreMesh(core_axis_name="c", subcore_axis_name="s")
    out_type = [
        jax.ShapeDtypeStruct((_E, 32), f32),
        jax.ShapeDtypeStruct((_E, 16), f32),
        jax.ShapeDtypeStruct((_E, 32), f32),
        jax.ShapeDtypeStruct((_E, 16), f32),
    ]

    @functools.partial(
        pl.kernel,
        out_type=out_type,
        mesh=mesh,
        compiler_params=pltpu.CompilerParams(use_tc_tiling_on_sc=False),
        scratch_types=[
            pltpu.VMEM((_CH,), jnp.int32),
            pltpu.VMEM((_CH,), jnp.int32),
            pltpu.VMEM((_CH,), jnp.int32),
            pltpu.VMEM((_CH,), jnp.int32),
            pltpu.VMEM((_CH, 32), f32),
            pltpu.VMEM((_CH, 16), f32),
            pltpu.VMEM((_CH, 32), f32),
            pltpu.VMEM((_CH, 16), f32),
            pltpu.SemaphoreType.DMA,
        ],
    )
    def k(ts_h, tx_h, iis_h, iid_h, ies_h, ied_h,
          o_is, o_id, o_es, o_ed,
          ix_a, ix_b, ix_c, ix_d, row_a, row_b, row_c, row_d, sem):
        wid = lax.axis_index("s") * 2 + lax.axis_index("c")
        nj = jnp.maximum(0, jnp.minimum(_NJ, _NROWS - wid * _NJ))

        def body(j, carry):
            base = (wid * _NJ + j) * _CH
            sl = pl.ds(base, _CH)
            c1 = [pltpu.async_copy(iis_h.at[sl], ix_a, sem),
                  pltpu.async_copy(iid_h.at[sl], ix_b, sem),
                  pltpu.async_copy(ies_h.at[sl], ix_c, sem),
                  pltpu.async_copy(ied_h.at[sl], ix_d, sem)]
            for c in c1:
                c.wait()
            c2 = [pltpu.async_copy(ts_h.at[ix_a], row_a, sem),
                  pltpu.async_copy(tx_h.at[ix_b], row_b, sem),
                  pltpu.async_copy(ts_h.at[ix_c], row_c, sem),
                  pltpu.async_copy(tx_h.at[ix_d], row_d, sem)]
            for c in c2:
                c.wait()
            c3 = [pltpu.async_copy(row_a, o_is.at[sl], sem),
                  pltpu.async_copy(row_b, o_id.at[sl], sem),
                  pltpu.async_copy(row_c, o_es.at[sl], sem),
                  pltpu.async_copy(row_d, o_ed.at[sl], sem)]
            for c in c3:
                c.wait()
            return carry

        lax.fori_loop(0, nj, body, 0)

    return k(ts, tx, ii_s, ii_d, ie_s, ie_d)


def kernel(X, E_in_idx, E_ex_idx, W_node, b_node, W_edge, b_edge,
           gain_nodes, bias_nodes, gain_edges, bias_edges):
    f32 = jnp.float32
    x12 = X.reshape(_N, 12)
    h_V, ts, tx = _node_call(x12, W_node, b_node.reshape(1, -1),
                             gain_nodes.reshape(1, -1), bias_nodes.reshape(1, -1))
    g_is, g_id, g_es, g_ed = _sc_gather(
        ts, tx, E_in_idx[0], E_in_idx[1], E_ex_idx[0], E_ex_idx[1])
    wd = W_edge[:256]
    wo = jnp.concatenate([W_edge[256:], jnp.zeros((4, 256), f32)], axis=0)
    be = b_edge.reshape(1, -1)
    ge = gain_edges.reshape(1, -1)
    bbe = bias_edges.reshape(1, -1)
    h_E_in = _edge_call(g_is, g_id, wd, wo, be, ge, bbe)
    h_E_ex = _edge_call(g_es, g_ed, wd, wo, be, ge, bbe)
    return h_V, h_E_in, h_E_ex
